# Initial kernel scaffold; baseline (speedup 1.0000x reference)
import jax
import jax.numpy as jnp
from jax.experimental import pallas as pl


def kernel(rpn_box_preds, rpn_cls_preds):
    raise NotImplementedError("write your pallas kernel here")


# speedup vs baseline: 7.6961x; 7.6961x over previous
import jax
import jax.numpy as jnp
from jax import lax
from jax.experimental import pallas as pl
from jax.experimental.pallas import tpu as pltpu

_PRE = 2048
_POST = 500
_THRESH = 0.7
_SUB = 8
_LANE = _PRE // _SUB


def _nms_body(rows_ref, feats_ref, out_ref, cnt_ref):
    out_ref[...] = jnp.zeros_like(out_ref)
    cnt_ref[0] = 0
    x1v = feats_ref[0, 0]
    x2v = feats_ref[0, 1]
    y1v = feats_ref[0, 2]
    y2v = feats_ref[0, 3]
    av = feats_ref[0, 4]
    lin = (lax.broadcasted_iota(jnp.int32, (_SUB, _LANE), 0) * _LANE
           + lax.broadcasted_iota(jnp.int32, (_SUB, _LANE), 1))

    def body(i, keep):
        row = rows_ref[0, pl.ds(i, 1), :]
        bx1 = row[0:1, 8:9]
        bx2 = row[0:1, 9:10]
        by1 = row[0:1, 10:11]
        by2 = row[0:1, 11:12]
        ba = row[0:1, 12:13]
        ix = jnp.maximum(jnp.minimum(bx2, x2v) - jnp.maximum(bx1, x1v), 0.0)
        iy = jnp.maximum(jnp.minimum(by2, y2v) - jnp.maximum(by1, y1v), 0.0)
        inter = ix * iy
        iou = inter / jnp.maximum(ba + av - inter, 1e-6)
        keep_i = jnp.max(jnp.where(lin == i, keep, 0.0))
        sup = (iou > _THRESH) & (lin > i)
        keep = jnp.where(sup, keep * (1.0 - keep_i), keep)
        cnt = cnt_ref[0]
        kept = keep_i > 0.5

        @pl.when(kept & (cnt < _POST))
        def _():
            out_ref[0, pl.ds(cnt, 1), :] = row

        cnt_ref[0] = cnt + kept.astype(jnp.int32)
        return keep

    lax.fori_loop(0, _PRE, body, jnp.ones((_SUB, _LANE), jnp.float32))


def kernel(rpn_box_preds, rpn_cls_preds):
    B = rpn_box_preds.shape[0]
    scores_all = jnp.max(rpn_cls_preds, axis=-1)
    labels_all = jnp.argmax(rpn_cls_preds, axis=-1)
    top_scores, top_idx = lax.top_k(scores_all, _PRE)
    tb = jnp.take_along_axis(rpn_box_preds, top_idx[..., None], axis=1)
    tl = jnp.take_along_axis(labels_all, top_idx, axis=1)

    x, y = tb[..., 0], tb[..., 1]
    dx, dy = tb[..., 3], tb[..., 4]
    x1 = x - dx * 0.5
    x2 = x + dx * 0.5
    y1 = y - dy * 0.5
    y2 = y + dy * 0.5
    area = dx * dy
    zc = jnp.zeros_like(x)
    rows = jnp.stack(
        [tb[..., 0], tb[..., 1], tb[..., 2], tb[..., 3], tb[..., 4],
         tb[..., 5], tb[..., 6], zc,
         x1, x2, y1, y2, area, top_scores,
         (tl + 1).astype(jnp.float32), zc], axis=-1)
    feats = jnp.stack(
        [x1.reshape(B, _SUB, _LANE), x2.reshape(B, _SUB, _LANE),
         y1.reshape(B, _SUB, _LANE), y2.reshape(B, _SUB, _LANE),
         area.reshape(B, _SUB, _LANE)], axis=1)

    out = pl.pallas_call(
        _nms_body,
        grid=(B,),
        in_specs=[
            pl.BlockSpec((1, _PRE, 16), lambda b: (b, 0, 0)),
            pl.BlockSpec((1, 5, _SUB, _LANE), lambda b: (b, 0, 0, 0)),
        ],
        out_specs=pl.BlockSpec((1, 512, 16), lambda b: (b, 0, 0)),
        out_shape=jax.ShapeDtypeStruct((B, 512, 16), jnp.float32),
        scratch_shapes=[pltpu.SMEM((1,), jnp.int32)],
    )(rows, feats)

    rois = out[:, :_POST, 0:7]
    roi_scores = out[:, :_POST, 13]
    roi_labels = out[:, :_POST, 14].astype(jnp.int32)
    return rois, roi_scores, roi_labels


# speedup vs baseline: 9.3078x; 1.2094x over previous
import jax
import jax.numpy as jnp
from jax import lax
from jax.experimental import pallas as pl

_PRE = 2048
_POST = 500
_THRESH = 0.7
_SUB = 8
_LANE = _PRE // _SUB
_G = 4
_NCOL = 24
_OUTS = 512


def _iou_row(r4, j, x1v, x2v, y1v, y2v, av):
    bx1 = r4[j:j + 1, 8:9]
    bx2 = r4[j:j + 1, 9:10]
    by1 = r4[j:j + 1, 10:11]
    by2 = r4[j:j + 1, 11:12]
    ba = r4[j:j + 1, 12:13]
    ix = jnp.maximum(jnp.minimum(bx2, x2v) - jnp.maximum(bx1, x1v), 0.0)
    iy = jnp.maximum(jnp.minimum(by2, y2v) - jnp.maximum(by1, y1v), 0.0)
    inter = ix * iy
    return inter / jnp.maximum(ba + av - inter, 1e-6)


def _nms_body(rows_ref, feats_ref, out_ref):
    x1v = feats_ref[0, 0]
    x2v = feats_ref[0, 1]
    y1v = feats_ref[0, 2]
    y2v = feats_ref[0, 3]
    av = feats_ref[0, 4]
    lin = (lax.broadcasted_iota(jnp.int32, (_SUB, _LANE), 0) * _LANE
           + lax.broadcasted_iota(jnp.int32, (_SUB, _LANE), 1))

    def group(g, keep):
        a0 = g * _G
        r4 = rows_ref[0, pl.ds(a0, _G), :]
        ious = [_iou_row(r4, j, x1v, x2v, y1v, y2v, av) for j in range(_G)]
        K = [jnp.max(jnp.where(lin == a0 + j, keep, 0.0), keepdims=True)
             for j in range(_G)]
        s01 = r4[0:1, 16:17]
        s02 = r4[0:1, 17:18]
        s03 = r4[0:1, 18:19]
        s12 = r4[1:2, 16:17]
        s13 = r4[1:2, 17:18]
        s23 = r4[2:3, 16:17]
        k0 = K[0]
        k1 = K[1] * (1.0 - s01 * k0)
        k2 = K[2] * (1.0 - s02 * k0) * (1.0 - s12 * k1)
        k3 = K[3] * (1.0 - s03 * k0) * (1.0 - s13 * k1) * (1.0 - s23 * k2)
        for j, kj in enumerate((k0, k1, k2, k3)):
            supf = ((ious[j] > _THRESH) & (lin > a0 + j)).astype(jnp.float32)
            keep = keep * (1.0 - supf * kj)
        return keep

    keep = lax.fori_loop(0, _PRE // _G, group,
                         jnp.ones((_SUB, _LANE), jnp.float32))

    iu = (lax.broadcasted_iota(jnp.int32, (_LANE, _LANE), 0)
          <= lax.broadcasted_iota(jnp.int32, (_LANE, _LANE), 1))
    cs = lax.dot(keep, iu.astype(jnp.float32))
    tot = cs[:, _LANE - 1:_LANE]
    lo = (lax.broadcasted_iota(jnp.int32, (_SUB, _SUB), 0)
          > lax.broadcasted_iota(jnp.int32, (_SUB, _SUB), 1))
    off = lax.dot(lo.astype(jnp.float32), tot)
    sel = cs + off - 1.0

    i512 = lax.broadcasted_iota(jnp.int32, (_OUTS, 128), 0)
    sel_i = sel.astype(jnp.int32)
    acc = jnp.zeros((_OUTS, _NCOL), jnp.float32)
    for c in range(_PRE // 128):
        r_, l0 = c // 2, (c % 2) * 128
        sel_s = sel_i[r_:r_ + 1, l0:l0 + 128]
        keep_s = keep[r_:r_ + 1, l0:l0 + 128]
        pt = (i512 == sel_s).astype(jnp.float32) * keep_s
        acc = acc + lax.dot(pt, rows_ref[0, c * 128:(c + 1) * 128, :],
                            precision=lax.Precision.HIGHEST)
    out_ref[0] = acc


def kernel(rpn_box_preds, rpn_cls_preds):
    B = rpn_box_preds.shape[0]
    scores_all = jnp.max(rpn_cls_preds, axis=-1)
    labels_all = jnp.argmax(rpn_cls_preds, axis=-1)
    top_scores, top_idx = lax.top_k(scores_all, _PRE)
    tb = jnp.take_along_axis(rpn_box_preds, top_idx[..., None], axis=1)
    tl = jnp.take_along_axis(labels_all, top_idx, axis=1)

    x, y = tb[..., 0], tb[..., 1]
    dx, dy = tb[..., 3], tb[..., 4]
    x1 = x - dx * 0.5
    x2 = x + dx * 0.5
    y1 = y - dy * 0.5
    y2 = y + dy * 0.5
    area = dx * dy

    def shift_sup(d):
        ix = jnp.maximum(jnp.minimum(x2[:, :-d], x2[:, d:])
                         - jnp.maximum(x1[:, :-d], x1[:, d:]), 0.0)
        iy = jnp.maximum(jnp.minimum(y2[:, :-d], y2[:, d:])
                         - jnp.maximum(y1[:, :-d], y1[:, d:]), 0.0)
        inter = ix * iy
        iou = inter / jnp.maximum(area[:, :-d] + area[:, d:] - inter, 1e-6)
        return jnp.pad((iou > _THRESH).astype(jnp.float32), ((0, 0), (0, d)))

    s1, s2, s3 = shift_sup(1), shift_sup(2), shift_sup(3)
    zc = jnp.zeros_like(x)
    rows = jnp.stack(
        [tb[..., 0], tb[..., 1], tb[..., 2], tb[..., 3], tb[..., 4],
         tb[..., 5], tb[..., 6], zc,
         x1, x2, y1, y2, area, top_scores,
         (tl + 1).astype(jnp.float32), zc,
         s1, s2, s3, zc, zc, zc, zc, zc], axis=-1)
    feats = jnp.stack(
        [x1.reshape(B, _SUB, _LANE), x2.reshape(B, _SUB, _LANE),
         y1.reshape(B, _SUB, _LANE), y2.reshape(B, _SUB, _LANE),
         area.reshape(B, _SUB, _LANE)], axis=1)

    out = pl.pallas_call(
        _nms_body,
        grid=(B,),
        in_specs=[
            pl.BlockSpec((1, _PRE, _NCOL), lambda b: (b, 0, 0)),
            pl.BlockSpec((1, 5, _SUB, _LANE), lambda b: (b, 0, 0, 0)),
        ],
        out_specs=pl.BlockSpec((1, _OUTS, _NCOL), lambda b: (b, 0, 0)),
        out_shape=jax.ShapeDtypeStruct((B, _OUTS, _NCOL), jnp.float32),
    )(rows, feats)

    rois = out[:, :_POST, 0:7]
    roi_scores = out[:, :_POST, 13]
    roi_labels = out[:, :_POST, 14].astype(jnp.int32)
    return rois, roi_scores, roi_labels


# speedup vs baseline: 36.7698x; 3.9504x over previous
import jax
import jax.numpy as jnp
from jax import lax
from jax.experimental import pallas as pl
from jax.experimental.pallas import tpu as pltpu

_PRE = 2048
_POST = 500
_THRESH = 0.7
_SUB = 8
_LANE = _PRE // _SUB
_NCOL = 24
_OUTS = 512


def _pair_sup(cx1, cx2, cy1, cy2, ca, rx1, rx2, ry1, ry2, ra):
    ix = jnp.maximum(jnp.minimum(cx2, rx2) - jnp.maximum(cx1, rx1), 0.0)
    iy = jnp.maximum(jnp.minimum(cy2, ry2) - jnp.maximum(cy1, ry1), 0.0)
    inter = ix * iy
    iou = inter / jnp.maximum(ca + ra - inter, 1e-6)
    return (iou > _THRESH).astype(jnp.float32)


def _nms_body(rows_ref, feats_ref, out_ref, keep_ref):
    x1v = feats_ref[0, 0]
    x2v = feats_ref[0, 1]
    y1v = feats_ref[0, 2]
    y2v = feats_ref[0, 3]
    av = feats_ref[0, 4]
    keep_ref[...] = jnp.ones((_SUB, _LANE), jnp.float32)

    for c in range(_SUB):
        base = c * _LANE
        fc = rows_ref[0, base:base + _LANE, :]
        cx1, cx2 = fc[:, 8:9], fc[:, 9:10]
        cy1, cy2 = fc[:, 10:11], fc[:, 11:12]
        ca = fc[:, 12:13]

        d = _pair_sup(cx1, cx2, cy1, cy2, ca,
                      x1v[c:c + 1, :], x2v[c:c + 1, :],
                      y1v[c:c + 1, :], y2v[c:c + 1, :], av[c:c + 1, :])
        tri = (lax.broadcasted_iota(jnp.int32, (_LANE, _LANE), 0)
               < lax.broadcasted_iota(jnp.int32, (_LANE, _LANE), 1))
        d = d * tri.astype(jnp.float32)

        keep0 = keep_ref[c:c + 1, :]

        def fix_cond(state):
            return state[1]

        def fix_body(state):
            k, _ = state
            a = lax.dot(k, d)
            k_new = keep0 * (a < 0.5).astype(jnp.float32)
            return k_new, jnp.any(k_new != k)

        k, _ = lax.while_loop(fix_cond, fix_body,
                              (keep0, jnp.bool_(True)))
        keep_ref[c:c + 1, :] = k

        for r in range(c, _SUB):
            for h in range(2):
                l0 = h * 128
                db = _pair_sup(cx1, cx2, cy1, cy2, ca,
                               x1v[r:r + 1, l0:l0 + 128],
                               x2v[r:r + 1, l0:l0 + 128],
                               y1v[r:r + 1, l0:l0 + 128],
                               y2v[r:r + 1, l0:l0 + 128],
                               av[r:r + 1, l0:l0 + 128])
                if r == c:
                    m = (lax.broadcasted_iota(jnp.int32, (_LANE, 128), 0)
                         < lax.broadcasted_iota(jnp.int32, (_LANE, 128), 1)
                         + l0)
                    db = db * m.astype(jnp.float32)
                sup = lax.dot(k, db)
                keep_ref[r:r + 1, l0:l0 + 128] = (
                    keep_ref[r:r + 1, l0:l0 + 128]
                    * (sup < 0.5).astype(jnp.float32))

    keep = keep_ref[...]
    iu = (lax.broadcasted_iota(jnp.int32, (_LANE, _LANE), 0)
          <= lax.broadcasted_iota(jnp.int32, (_LANE, _LANE), 1))
    cs = lax.dot(keep, iu.astype(jnp.float32))
    tot = cs[:, _LANE - 1:_LANE]
    lo = (lax.broadcasted_iota(jnp.int32, (_SUB, _SUB), 0)
          > lax.broadcasted_iota(jnp.int32, (_SUB, _SUB), 1))
    off = lax.dot(lo.astype(jnp.float32), tot)
    sel = cs + off - 1.0

    i512 = lax.broadcasted_iota(jnp.int32, (_OUTS, 128), 0)
    sel_i = sel.astype(jnp.int32)
    acc = jnp.zeros((_OUTS, _NCOL), jnp.float32)
    for c in range(_PRE // 128):
        r_, l0 = c // 2, (c % 2) * 128
        sel_s = sel_i[r_:r_ + 1, l0:l0 + 128]
        keep_s = keep[r_:r_ + 1, l0:l0 + 128]
        pt = (i512 == sel_s).astype(jnp.float32) * keep_s
        acc = acc + lax.dot(pt, rows_ref[0, c * 128:(c + 1) * 128, :],
                            precision=lax.Precision.HIGHEST)
    out_ref[0] = acc


def kernel(rpn_box_preds, rpn_cls_preds):
    B = rpn_box_preds.shape[0]
    scores_all = jnp.max(rpn_cls_preds, axis=-1)
    labels_all = jnp.argmax(rpn_cls_preds, axis=-1)
    top_scores, top_idx = lax.top_k(scores_all, _PRE)
    tb = jnp.take_along_axis(rpn_box_preds, top_idx[..., None], axis=1)
    tl = jnp.take_along_axis(labels_all, top_idx, axis=1)

    x, y = tb[..., 0], tb[..., 1]
    dx, dy = tb[..., 3], tb[..., 4]
    x1 = x - dx * 0.5
    x2 = x + dx * 0.5
    y1 = y - dy * 0.5
    y2 = y + dy * 0.5
    area = dx * dy
    zc = jnp.zeros_like(x)
    rows = jnp.stack(
        [tb[..., 0], tb[..., 1], tb[..., 2], tb[..., 3], tb[..., 4],
         tb[..., 5], tb[..., 6], zc,
         x1, x2, y1, y2, area, top_scores,
         (tl + 1).astype(jnp.float32), zc,
         zc, zc, zc, zc, zc, zc, zc, zc], axis=-1)
    feats = jnp.stack(
        [x1.reshape(B, _SUB, _LANE), x2.reshape(B, _SUB, _LANE),
         y1.reshape(B, _SUB, _LANE), y2.reshape(B, _SUB, _LANE),
         area.reshape(B, _SUB, _LANE)], axis=1)

    out = pl.pallas_call(
        _nms_body,
        grid=(B,),
        in_specs=[
            pl.BlockSpec((1, _PRE, _NCOL), lambda b: (b, 0, 0)),
            pl.BlockSpec((1, 5, _SUB, _LANE), lambda b: (b, 0, 0, 0)),
        ],
        out_specs=pl.BlockSpec((1, _OUTS, _NCOL), lambda b: (b, 0, 0)),
        out_shape=jax.ShapeDtypeStruct((B, _OUTS, _NCOL), jnp.float32),
        scratch_shapes=[pltpu.VMEM((_SUB, _LANE), jnp.float32)],
    )(rows, feats)

    rois = out[:, :_POST, 0:7]
    roi_scores = out[:, :_POST, 13]
    roi_labels = out[:, :_POST, 14].astype(jnp.int32)
    return rois, roi_scores, roi_labels
